# baseline (speedup 1.0000x reference)
import functools

import jax
import jax.numpy as jnp
from jax import lax
from jax.experimental import pallas as pl
from jax.experimental.pallas import tpu as pltpu
from jax.experimental.pallas import tpu_sc as plsc

_NC = 2
_NS = 16
_NW = _NC * _NS
_CHUNK = 128
_LANES = 16


@functools.lru_cache(maxsize=None)
def _build(n_env, buf, batch, n_obs, n_act, n_cobs):
    total = n_env * batch
    per_w = total // _NW
    n_chunk = per_w // _CHUNK
    env_per_w = per_w // batch
    chunks_per_env = batch // _CHUNK

    mesh = plsc.VectorSubcoreMesh(core_axis_name="c", subcore_axis_name="s")

    out_type = (
        jax.ShapeDtypeStruct((total, n_obs), jnp.float32),
        jax.ShapeDtypeStruct((total, n_act), jnp.float32),
        jax.ShapeDtypeStruct((total,), jnp.float32),
        jax.ShapeDtypeStruct((total,), jnp.int32),
        jax.ShapeDtypeStruct((total,), jnp.int32),
        jax.ShapeDtypeStruct((total, n_obs), jnp.float32),
        jax.ShapeDtypeStruct((total, n_cobs), jnp.float32),
        jax.ShapeDtypeStruct((total, n_cobs), jnp.float32),
    )
    def _buf_set():
        return [
            pltpu.VMEM((_CHUNK, n_obs), jnp.float32),
            pltpu.VMEM((_CHUNK, n_obs), jnp.float32),
            pltpu.VMEM((_CHUNK, n_cobs), jnp.float32),
            pltpu.VMEM((_CHUNK, n_cobs), jnp.float32),
            pltpu.VMEM((_CHUNK, n_act), jnp.float32),
            pltpu.VMEM((_CHUNK,), jnp.float32),
            pltpu.VMEM((_CHUNK,), jnp.int32),
            pltpu.VMEM((_CHUNK,), jnp.int32),
        ]

    scratch_types = (
        [pltpu.VMEM((n_chunk, _CHUNK), jnp.int32)]
        + _buf_set() + _buf_set()
        + [pltpu.SemaphoreType.DMA] * 4
    )

    @functools.partial(pl.kernel, out_type=out_type, mesh=mesh,
                       scratch_types=scratch_types,
                       compiler_params=pltpu.CompilerParams(
                           use_tc_tiling_on_sc=False))
    def k(obs_h, act_h, rew_h, nobs_h, cobs_h, ncobs_h, don_h, trc_h, idx_h,
          o_obs, o_act, o_rew, o_don, o_trc, o_nobs, o_cobs, o_ncobs,
          idx_v,
          a0, a1, a2, a3, a4, a5, a6, a7,
          b0, b1, b2, b3, b4, b5, b6, b7,
          gsem0, gsem1, ssem0, ssem1):
        wid = lax.axis_index("s") * _NC + lax.axis_index("c")
        pltpu.sync_copy(idx_h.at[pl.ds(wid * n_chunk, n_chunk)], idx_v)

        def add_body(c, carry):
            env = wid * env_per_w + c // chunks_per_env
            base = env * buf
            for j in range(_CHUNK // _LANES):
                sl = pl.ds(j * _LANES, _LANES)
                idx_v[c, sl] = idx_v[c, sl] + base
            return carry

        lax.fori_loop(0, n_chunk, add_body, 0)

        tables = (obs_h, nobs_h, cobs_h, ncobs_h, act_h, rew_h, don_h, trc_h)
        outs = (o_obs, o_nobs, o_cobs, o_ncobs, o_act, o_rew, o_don, o_trc)
        bufs = ((a0, a1, a2, a3, a4, a5, a6, a7),
                (b0, b1, b2, b3, b4, b5, b6, b7))
        gsems = (gsem0, gsem1)
        ssems = (ssem0, ssem1)

        def gathers(c):
            s = c % 2
            rows = idx_v.at[c]
            return [pltpu.make_async_copy(t.at[rows], b, gsems[s])
                    for t, b in zip(tables, bufs[s])]

        def stores(c):
            s = c % 2
            dsl = pl.ds(wid * per_w + c * _CHUNK, _CHUNK)
            return [pltpu.make_async_copy(b, o.at[dsl], ssems[s])
                    for b, o in zip(bufs[s], outs)]

        pending_stores = [None, None]
        for c in range(n_chunk):
            s = c % 2
            if pending_stores[s] is not None:
                for cp in pending_stores[s]:
                    cp.wait()
                pending_stores[s] = None
            g = gathers(c)
            for cp in g:
                cp.start()
            if c > 0:
                prev = c - 1
                for cp in pending_gathers:
                    cp.wait()
                st = stores(prev)
                for cp in st:
                    cp.start()
                pending_stores[prev % 2] = st
            pending_gathers = g
        for cp in pending_gathers:
            cp.wait()
        st = stores(n_chunk - 1)
        for cp in st:
            cp.start()
        pending_stores[(n_chunk - 1) % 2] = st
        for ps in pending_stores:
            if ps is not None:
                for cp in ps:
                    cp.wait()

    return k


def kernel(observations, actions, rewards, next_observations,
           critic_observations, next_critic_observations, dones, truncations,
           indices, batch_size):
    n_env, buf, n_obs = observations.shape
    n_act = actions.shape[2]
    n_cobs = critic_observations.shape[2]
    batch = indices.shape[1]
    total = n_env * batch

    k = _build(n_env, buf, batch, n_obs, n_act, n_cobs)
    obs_s, act_s, rew_s, don_s, trc_s, nobs_s, cobs_s, ncobs_s = k(
        observations.reshape(n_env * buf, n_obs),
        actions.reshape(n_env * buf, n_act),
        rewards.reshape(n_env * buf),
        next_observations.reshape(n_env * buf, n_obs),
        critic_observations.reshape(n_env * buf, n_cobs),
        next_critic_observations.reshape(n_env * buf, n_cobs),
        dones.reshape(n_env * buf),
        truncations.reshape(n_env * buf),
        indices.reshape(total // _CHUNK, _CHUNK),
    )
    don_s = don_s + jnp.asarray(batch_size, dtype=jnp.int32) * 0
    eff_n = jnp.ones((total,), dtype=jnp.int32)
    return (obs_s, act_s, rew_s, don_s, trc_s, nobs_s, eff_n, cobs_s, ncobs_s)
